# baseline (speedup 1.0000x reference)
import functools
import jax
import jax.numpy as jnp
from jax import lax
from jax.experimental import pallas as pl
from jax.experimental.pallas import tpu as pltpu
from jax.experimental.pallas import tpu_sc as plsc

_EPS = 0.1
_V = 100000
_N = 2048
_FILL = _EPS / (_V - 2)

_NTC = 1280
_NSC = _N - _NTC
_VSC = 96000
_VTAIL = _V - _VSC

_VR = 16

_NCORES = 2
_NSUB = 16
_NW = _NCORES * _NSUB
_TRW = _NSC // (8 * _NW)
_RPW = 8 * _TRW
_CW = 6400
_NCH = _VSC // _CW
_VPR = _CW // 16
_GRP = 20
_NGRP = _VPR // _GRP


def _tc_stats_kernel(pred_ref, tgt_ref, m_ref, s_ref, tot_ref, tv_ref):
    x = pred_ref[...]
    m = jnp.max(x, axis=1, keepdims=True)
    tot = jnp.sum(x, axis=1, keepdims=True)
    s = jnp.sum(jnp.exp(x - m), axis=1, keepdims=True)
    cols = lax.broadcasted_iota(jnp.int32, x.shape, 1)
    tv = jnp.sum(jnp.where(cols == tgt_ref[...], x, 0.0), axis=1,
                 keepdims=True)
    m_ref[...] = m
    s_ref[...] = s
    tot_ref[...] = tot
    tv_ref[...] = tv


def _tc_tail_kernel(pred_ref, tgt_ref, m_ref, s_ref, tot_ref, tv_ref):
    x = pred_ref[...]
    m = jnp.max(x, axis=1, keepdims=True)
    tot = jnp.sum(x, axis=1, keepdims=True)
    s = jnp.sum(jnp.exp(x - m), axis=1, keepdims=True)
    cols = lax.broadcasted_iota(jnp.int32, x.shape, 1) + _VSC
    tv = jnp.sum(jnp.where(cols == tgt_ref[...], x, 0.0), axis=1,
                 keepdims=True)
    m_ref[...] = m
    s_ref[...] = s
    tot_ref[...] = tot
    tv_ref[...] = tv


@functools.cache
def _make_sc_stats():
    mesh = plsc.VectorSubcoreMesh(core_axis_name="c", subcore_axis_name="s")
    st = jax.ShapeDtypeStruct((_NSC * 16,), jnp.float32)

    @functools.partial(
        pl.kernel,
        mesh=mesh,
        out_type=[st, st, st, st],
        scratch_types=[
            pltpu.VMEM((8, _CW), jnp.float32),
            pltpu.VMEM((8, _CW), jnp.float32),
            pltpu.VMEM((((_RPW + 15) // 16) * 16,), jnp.int32),
            pltpu.VMEM((_RPW * 16,), jnp.float32),
            pltpu.VMEM((_RPW * 16,), jnp.float32),
            pltpu.VMEM((_RPW * 16,), jnp.float32),
            pltpu.VMEM((_RPW * 16,), jnp.float32),
            pltpu.SemaphoreType.DMA,
            pltpu.SemaphoreType.DMA,
        ],
        compiler_params=pltpu.CompilerParams(use_tc_tiling_on_sc=True,
                                             needs_layout_passes=False),
    )
    def _sc_stats(pred_hbm, tgt_hbm, m_hbm, s_hbm, tot_hbm, tv_hbm,
                  buf0, buf1, tgt_v, m_st, s_st, tot_st, tv_st, sem0, sem1):
        wid = lax.axis_index("s") * _NCORES + lax.axis_index("c")
        row0 = wid * _RPW
        out0 = wid * _RPW
        bufs = (buf0, buf1)
        sems = (sem0, sem1)

        pltpu.sync_copy(tgt_hbm.at[pl.ds(row0, _RPW)], tgt_v.at[pl.ds(0, _RPW)])

        neg = jnp.full((16,), -1e30, jnp.float32)
        zero = jnp.zeros((16,), jnp.float32)
        for rl in range(_RPW):
            m_st[pl.ds(rl * 16, 16)] = neg
            s_st[pl.ds(rl * 16, 16)] = zero
            tot_st[pl.ds(rl * 16, 16)] = zero
            tv_st[pl.ds(rl * 16, 16)] = zero

        lane_iota = lax.iota(jnp.int32, 16)

        def _src(tr, q):
            return pred_hbm.at[pl.ds(row0 + tr * 8, 8), pl.ds(q * _CW, _CW)]

        def _chunk(tr, q, buf):
            colbase = q * _CW
            for r in range(8):
                rl = tr * 8 + r
                soff = pl.multiple_of(rl * 16, 16)
                tvec = tgt_v[pl.ds(pl.multiple_of((rl // 16) * 16, 16), 16)]
                t_r = jnp.sum(jnp.where(lane_iota == (rl % 16), tvec, 0))
                m16 = m_st[pl.ds(soff, 16)]
                s16 = s_st[pl.ds(soff, 16)]
                tot16 = tot_st[pl.ds(soff, 16)]
                tv16 = tv_st[pl.ds(soff, 16)]

                def _pass_a(k, carry):
                    cmax, tot, tv = carry
                    base = k * (_GRP * 16)
                    for u in range(_GRP):
                        off = base + u * 16
                        v = buf[r, pl.ds(off, 16)]
                        col = lane_iota + (colbase + off)
                        cmax = jnp.maximum(cmax, v)
                        tot = tot + v
                        tv = tv + jnp.where(col == t_r, v, 0.0)
                    return cmax, tot, tv

                cmax, tot16, tv16 = lax.fori_loop(
                    0, _NGRP, _pass_a, (neg, tot16, tv16))
                m_new = jnp.maximum(m16, cmax)
                s16 = s16 * jnp.exp(m16 - m_new)

                def _pass_b(k, s):
                    base = k * (_GRP * 16)
                    for u in range(_GRP):
                        v = buf[r, pl.ds(base + u * 16, 16)]
                        s = s + jnp.exp(v - m_new)
                    return s

                s16 = lax.fori_loop(0, _NGRP, _pass_b, s16)
                m_st[pl.ds(soff, 16)] = m_new
                s_st[pl.ds(soff, 16)] = s16
                tot_st[pl.ds(soff, 16)] = tot16
                tv_st[pl.ds(soff, 16)] = tv16

        def _tile_row(tr, carry):
            pltpu.async_copy(_src(tr, 0), buf0, sem0)
            pltpu.async_copy(_src(tr, 1), buf1, sem1)

            def _ring(g, carry2):
                for b in range(2):
                    q = g * 2 + b

                    @pl.when(q < _NCH)
                    def _():
                        pltpu.make_async_copy(
                            _src(tr, q), bufs[b], sems[b]).wait()
                        _chunk(tr, q, bufs[b])

                        @pl.when(q + 2 < _NCH)
                        def _():
                            pltpu.async_copy(_src(tr, q + 2), bufs[b], sems[b])
                return carry2

            lax.fori_loop(0, (_NCH + 2) // 2, _ring, 0)
            return carry

        lax.fori_loop(0, _TRW, _tile_row, 0)

        pltpu.sync_copy(m_st, m_hbm.at[pl.ds(out0 * 16, _RPW * 16)])
        pltpu.sync_copy(s_st, s_hbm.at[pl.ds(out0 * 16, _RPW * 16)])
        pltpu.sync_copy(tot_st, tot_hbm.at[pl.ds(out0 * 16, _RPW * 16)])
        pltpu.sync_copy(tv_st, tv_hbm.at[pl.ds(out0 * 16, _RPW * 16)])

    return _sc_stats


def _combine_kernel(am_ref, as_ref, atot_ref, atv_ref,
                    scm_ref, scs_ref, sctot_ref, sctv_ref,
                    tm_ref, ts_ref, ttot_ref, ttv_ref,
                    tgt_ref, out_ref):
    def _loss(logz, tot, tv, tgt):
        s_row = tot - _V * logz
        logp_t = tv - logz
        loss = -(_FILL * s_row + (1.0 - _EPS - _FILL) * logp_t)
        mask = tgt != 0
        lsum = jnp.sum(jnp.where(mask, loss, 0.0), keepdims=True)
        cnt = jnp.sum(mask.astype(jnp.float32), keepdims=True)
        return lsum.reshape(1, 1), cnt.reshape(1, 1)

    logz_a = am_ref[...] + jnp.log(as_ref[...])
    l1, c1 = _loss(logz_a, atot_ref[...], atv_ref[...], tgt_ref[0:_NTC, :])

    scm = scm_ref[...]
    mb = jnp.max(scm, axis=1, keepdims=True)
    sb = jnp.sum(scs_ref[...] * jnp.exp(scm - mb), axis=1, keepdims=True)
    tm = tm_ref[...]
    mm = jnp.maximum(mb, tm)
    ss = sb * jnp.exp(mb - mm) + ts_ref[...] * jnp.exp(tm - mm)
    tot = jnp.sum(sctot_ref[...], axis=1, keepdims=True) + ttot_ref[...]
    tv = jnp.sum(sctv_ref[...], axis=1, keepdims=True) + ttv_ref[...]
    logz_b = mm + jnp.log(ss)
    l2, c2 = _loss(logz_b, tot, tv, tgt_ref[_NTC:_N, :])

    lsum = l1 + l2
    cnt = c1 + c2
    out_ref[...] = jnp.where(cnt > 0, lsum / jnp.maximum(cnt, 1.0), 0.0)


def kernel(pred, target):
    tgt2 = target.reshape(_N, 1)

    pred_sc = lax.slice(pred, (_NTC, 0), (_N, _V))
    tgt_sc = lax.slice(target, (_NTC,), (_N,))
    scm, scs, sctot, sctv = (a.reshape(_NSC, 16)
                             for a in _make_sc_stats()(pred_sc, tgt_sc))

    am, a_s, atot, atv = pl.pallas_call(
        _tc_stats_kernel,
        grid=(_NTC // _VR,),
        in_specs=[
            pl.BlockSpec((_VR, _V), lambda i: (i, 0)),
            pl.BlockSpec((_VR, 1), lambda i: (i, 0)),
        ],
        out_specs=[pl.BlockSpec((_VR, 1), lambda i: (i, 0))] * 4,
        out_shape=[jax.ShapeDtypeStruct((_NTC, 1), jnp.float32)] * 4,
    )(pred, tgt2)

    tail = lax.slice(pred, (_NTC, _VSC), (_N, _V))
    tm, ts, ttot, ttv = pl.pallas_call(
        _tc_tail_kernel,
        grid=(_NSC // 64,),
        in_specs=[
            pl.BlockSpec((64, _VTAIL), lambda i: (i, 0)),
            pl.BlockSpec((64, 1), lambda i: (_NTC // 64 + i, 0)),
        ],
        out_specs=[pl.BlockSpec((64, 1), lambda i: (i, 0))] * 4,
        out_shape=[jax.ShapeDtypeStruct((_NSC, 1), jnp.float32)] * 4,
    )(tail, tgt2)

    out = pl.pallas_call(
        _combine_kernel,
        out_shape=jax.ShapeDtypeStruct((1, 1), jnp.float32),
    )(am, a_s, atot, atv, scm, scs, sctot, sctv, tm, ts, ttot, ttv, tgt2)
    return out[0, 0]


# speedup vs baseline: 1.1148x; 1.1148x over previous
import functools
import jax
import jax.numpy as jnp
from jax import lax
from jax.experimental import pallas as pl
from jax.experimental.pallas import tpu as pltpu
from jax.experimental.pallas import tpu_sc as plsc

_EPS = 0.1
_V = 100000
_N = 2048
_FILL = _EPS / (_V - 2)

_VSC = 96000
_VTAIL = _V - _VSC

_VR = 16

_NCORES = 2
_NSUB = 16
_NW = _NCORES * _NSUB
_RPW = _N // _NW
_NCH = _RPW // 8
_VPR = _VTAIL // 16
_GRP = 25
_NGRP = _VPR // _GRP


def _tc_stats_kernel(pred_ref, tgt_ref, m_ref, s_ref, tot_ref, tv_ref):
    x = pred_ref[...]
    m = jnp.max(x, axis=1, keepdims=True)
    tot = jnp.sum(x, axis=1, keepdims=True)
    s = jnp.sum(jnp.exp(x - m), axis=1, keepdims=True)
    cols = lax.broadcasted_iota(jnp.int32, x.shape, 1)
    tv = jnp.sum(jnp.where(cols == tgt_ref[...], x, 0.0), axis=1,
                 keepdims=True)
    m_ref[...] = m
    s_ref[...] = s
    tot_ref[...] = tot
    tv_ref[...] = tv


@functools.cache
def _make_sc_stats():
    mesh = plsc.VectorSubcoreMesh(core_axis_name="c", subcore_axis_name="s")
    st = jax.ShapeDtypeStruct((_N * 16,), jnp.float32)

    @functools.partial(
        pl.kernel,
        mesh=mesh,
        out_type=[st, st, st, st],
        scratch_types=[
            pltpu.VMEM((8, _VTAIL), jnp.float32),
            pltpu.VMEM((8, _VTAIL), jnp.float32),
            pltpu.VMEM((_RPW,), jnp.int32),
            pltpu.VMEM((_RPW * 16,), jnp.float32),
            pltpu.VMEM((_RPW * 16,), jnp.float32),
            pltpu.VMEM((_RPW * 16,), jnp.float32),
            pltpu.VMEM((_RPW * 16,), jnp.float32),
            pltpu.SemaphoreType.DMA,
            pltpu.SemaphoreType.DMA,
        ],
        compiler_params=pltpu.CompilerParams(use_tc_tiling_on_sc=True,
                                             needs_layout_passes=False),
    )
    def _sc_stats(tail_hbm, tgt_hbm, m_hbm, s_hbm, tot_hbm, tv_hbm,
                  buf0, buf1, tgt_v, m_st, s_st, tot_st, tv_st, sem0, sem1):
        wid = lax.axis_index("s") * _NCORES + lax.axis_index("c")
        row0 = wid * _RPW
        bufs = (buf0, buf1)
        sems = (sem0, sem1)

        pltpu.sync_copy(tgt_hbm.at[pl.ds(row0, _RPW)], tgt_v)

        neg = jnp.full((16,), -1e30, jnp.float32)
        zero = jnp.zeros((16,), jnp.float32)
        for rl in range(_RPW):
            m_st[pl.ds(rl * 16, 16)] = neg
            s_st[pl.ds(rl * 16, 16)] = zero
            tot_st[pl.ds(rl * 16, 16)] = zero
            tv_st[pl.ds(rl * 16, 16)] = zero

        lane_iota = lax.iota(jnp.int32, 16)

        def _src(q):
            return tail_hbm.at[pl.ds(row0 + q * 8, 8), pl.ds(0, _VTAIL)]

        def _chunk(q, buf):
            for r in range(8):
                rl = q * 8 + r
                soff = pl.multiple_of(rl * 16, 16)
                tvec = tgt_v[pl.ds(pl.multiple_of((rl // 16) * 16, 16), 16)]
                t_r = jnp.sum(jnp.where(lane_iota == (rl % 16), tvec, 0))
                m16 = m_st[pl.ds(soff, 16)]
                s16 = s_st[pl.ds(soff, 16)]
                tot16 = tot_st[pl.ds(soff, 16)]
                tv16 = tv_st[pl.ds(soff, 16)]

                def _pass_a(k, carry):
                    cmax, tot, tv = carry
                    base = k * (_GRP * 16)
                    for u in range(_GRP):
                        off = base + u * 16
                        v = buf[r, pl.ds(off, 16)]
                        col = lane_iota + (_VSC + off)
                        cmax = jnp.maximum(cmax, v)
                        tot = tot + v
                        tv = tv + jnp.where(col == t_r, v, 0.0)
                    return cmax, tot, tv

                cmax, tot16, tv16 = lax.fori_loop(
                    0, _NGRP, _pass_a, (neg, tot16, tv16))
                m_new = jnp.maximum(m16, cmax)
                s16 = s16 * jnp.exp(m16 - m_new)

                def _pass_b(k, s):
                    base = k * (_GRP * 16)
                    for u in range(_GRP):
                        v = buf[r, pl.ds(base + u * 16, 16)]
                        s = s + jnp.exp(v - m_new)
                    return s

                s16 = lax.fori_loop(0, _NGRP, _pass_b, s16)
                m_st[pl.ds(soff, 16)] = m_new
                s_st[pl.ds(soff, 16)] = s16
                tot_st[pl.ds(soff, 16)] = tot16
                tv_st[pl.ds(soff, 16)] = tv16

        pltpu.async_copy(_src(0), buf0, sem0)
        pltpu.async_copy(_src(1), buf1, sem1)

        def _ring(g, carry):
            for b in range(2):
                q = g * 2 + b

                @pl.when(q < _NCH)
                def _():
                    pltpu.make_async_copy(_src(q), bufs[b], sems[b]).wait()
                    _chunk(q, bufs[b])

                    @pl.when(q + 2 < _NCH)
                    def _():
                        pltpu.async_copy(_src(q + 2), bufs[b], sems[b])
            return carry

        lax.fori_loop(0, (_NCH + 1) // 2, _ring, 0)

        pltpu.sync_copy(m_st, m_hbm.at[pl.ds(row0 * 16, _RPW * 16)])
        pltpu.sync_copy(s_st, s_hbm.at[pl.ds(row0 * 16, _RPW * 16)])
        pltpu.sync_copy(tot_st, tot_hbm.at[pl.ds(row0 * 16, _RPW * 16)])
        pltpu.sync_copy(tv_st, tv_hbm.at[pl.ds(row0 * 16, _RPW * 16)])

    return _sc_stats


def _combine_kernel(am_ref, as_ref, atot_ref, atv_ref,
                    scm_ref, scs_ref, sctot_ref, sctv_ref,
                    tgt_ref, out_ref):
    scm = scm_ref[...]
    mb = jnp.max(scm, axis=1, keepdims=True)
    sb = jnp.sum(scs_ref[...] * jnp.exp(scm - mb), axis=1, keepdims=True)
    ma = am_ref[...]
    mm = jnp.maximum(ma, mb)
    ss = as_ref[...] * jnp.exp(ma - mm) + sb * jnp.exp(mb - mm)
    tot = atot_ref[...] + jnp.sum(sctot_ref[...], axis=1, keepdims=True)
    tv = atv_ref[...] + jnp.sum(sctv_ref[...], axis=1, keepdims=True)
    logz = mm + jnp.log(ss)
    s_row = tot - _V * logz
    logp_t = tv - logz
    loss = -(_FILL * s_row + (1.0 - _EPS - _FILL) * logp_t)
    mask = tgt_ref[...] != 0
    lsum = jnp.sum(jnp.where(mask, loss, 0.0), keepdims=True).reshape(1, 1)
    cnt = jnp.sum(mask.astype(jnp.float32), keepdims=True).reshape(1, 1)
    out_ref[...] = jnp.where(cnt > 0, lsum / jnp.maximum(cnt, 1.0), 0.0)


def kernel(pred, target):
    tgt2 = target.reshape(_N, 1)

    tail = lax.slice(pred, (0, _VSC), (_N, _V))
    scm, scs, sctot, sctv = (a.reshape(_N, 16)
                             for a in _make_sc_stats()(tail, target))

    am, a_s, atot, atv = pl.pallas_call(
        _tc_stats_kernel,
        grid=(_N // _VR,),
        in_specs=[
            pl.BlockSpec((_VR, _VSC), lambda i: (i, 0)),
            pl.BlockSpec((_VR, 1), lambda i: (i, 0)),
        ],
        out_specs=[pl.BlockSpec((_VR, 1), lambda i: (i, 0))] * 4,
        out_shape=[jax.ShapeDtypeStruct((_N, 1), jnp.float32)] * 4,
    )(pred, tgt2)

    out = pl.pallas_call(
        _combine_kernel,
        out_shape=jax.ShapeDtypeStruct((1, 1), jnp.float32),
    )(am, a_s, atot, atv, scm, scs, sctot, sctv, tgt2)
    return out[0, 0]


# speedup vs baseline: 1.1727x; 1.0520x over previous
import functools
import jax
import jax.numpy as jnp
from jax import lax
from jax.experimental import pallas as pl
from jax.experimental.pallas import tpu as pltpu
from jax.experimental.pallas import tpu_sc as plsc

_EPS = 0.1
_V = 100000
_N = 2048
_FILL = _EPS / (_V - 2)

_VSC = 96000
_VTAIL = _V - _VSC

_VR = 32

_NCORES = 2
_NSUB = 16
_NW = _NCORES * _NSUB
_RPW = _N // _NW
_NCH = _RPW // 8
_VPR = _VTAIL // 16
_GRP = 25
_NGRP = _VPR // _GRP


def _tc_stats_kernel(pred_ref, tgt_ref, m_ref, s_ref, tot_ref, tv_ref):
    x = pred_ref[...]
    m = jnp.max(x, axis=1, keepdims=True)
    tot = jnp.sum(x, axis=1, keepdims=True)
    s = jnp.sum(jnp.exp(x - m), axis=1, keepdims=True)
    cols = lax.broadcasted_iota(jnp.int32, x.shape, 1)
    tv = jnp.sum(jnp.where(cols == tgt_ref[...], x, 0.0), axis=1,
                 keepdims=True)
    m_ref[...] = m
    s_ref[...] = s
    tot_ref[...] = tot
    tv_ref[...] = tv


@functools.cache
def _make_sc_stats():
    mesh = plsc.VectorSubcoreMesh(core_axis_name="c", subcore_axis_name="s")
    st = jax.ShapeDtypeStruct((_N * 16,), jnp.float32)

    @functools.partial(
        pl.kernel,
        mesh=mesh,
        out_type=[st, st, st, st],
        scratch_types=[
            pltpu.VMEM((8, _VTAIL), jnp.float32),
            pltpu.VMEM((8, _VTAIL), jnp.float32),
            pltpu.VMEM((_RPW,), jnp.int32),
            pltpu.VMEM((_RPW * 16,), jnp.float32),
            pltpu.VMEM((_RPW * 16,), jnp.float32),
            pltpu.VMEM((_RPW * 16,), jnp.float32),
            pltpu.VMEM((_RPW * 16,), jnp.float32),
            pltpu.SemaphoreType.DMA,
            pltpu.SemaphoreType.DMA,
        ],
        compiler_params=pltpu.CompilerParams(use_tc_tiling_on_sc=True,
                                             needs_layout_passes=False),
    )
    def _sc_stats(tail_hbm, tgt_hbm, m_hbm, s_hbm, tot_hbm, tv_hbm,
                  buf0, buf1, tgt_v, m_st, s_st, tot_st, tv_st, sem0, sem1):
        wid = lax.axis_index("s") * _NCORES + lax.axis_index("c")
        row0 = wid * _RPW
        bufs = (buf0, buf1)
        sems = (sem0, sem1)

        pltpu.sync_copy(tgt_hbm.at[pl.ds(row0, _RPW)], tgt_v)

        neg = jnp.full((16,), -1e30, jnp.float32)
        zero = jnp.zeros((16,), jnp.float32)
        for rl in range(_RPW):
            m_st[pl.ds(rl * 16, 16)] = neg
            s_st[pl.ds(rl * 16, 16)] = zero
            tot_st[pl.ds(rl * 16, 16)] = zero
            tv_st[pl.ds(rl * 16, 16)] = zero

        lane_iota = lax.iota(jnp.int32, 16)

        def _src(q):
            return tail_hbm.at[pl.ds(row0 + q * 8, 8), pl.ds(0, _VTAIL)]

        def _chunk(q, buf):
            for r in range(8):
                rl = q * 8 + r
                soff = pl.multiple_of(rl * 16, 16)
                tvec = tgt_v[pl.ds(pl.multiple_of((rl // 16) * 16, 16), 16)]
                t_r = jnp.sum(jnp.where(lane_iota == (rl % 16), tvec, 0))
                m16 = m_st[pl.ds(soff, 16)]
                s16 = s_st[pl.ds(soff, 16)]
                tot16 = tot_st[pl.ds(soff, 16)]
                tv16 = tv_st[pl.ds(soff, 16)]

                def _pass_a(k, carry):
                    cmax, tot, tv = carry
                    base = k * (_GRP * 16)
                    for u in range(_GRP):
                        off = base + u * 16
                        v = buf[r, pl.ds(off, 16)]
                        col = lane_iota + (_VSC + off)
                        cmax = jnp.maximum(cmax, v)
                        tot = tot + v
                        tv = tv + jnp.where(col == t_r, v, 0.0)
                    return cmax, tot, tv

                cmax, tot16, tv16 = lax.fori_loop(
                    0, _NGRP, _pass_a, (neg, tot16, tv16))
                m_new = jnp.maximum(m16, cmax)
                s16 = s16 * jnp.exp(m16 - m_new)

                def _pass_b(k, s):
                    base = k * (_GRP * 16)
                    for u in range(_GRP):
                        v = buf[r, pl.ds(base + u * 16, 16)]
                        s = s + jnp.exp(v - m_new)
                    return s

                s16 = lax.fori_loop(0, _NGRP, _pass_b, s16)
                m_st[pl.ds(soff, 16)] = m_new
                s_st[pl.ds(soff, 16)] = s16
                tot_st[pl.ds(soff, 16)] = tot16
                tv_st[pl.ds(soff, 16)] = tv16

        pltpu.async_copy(_src(0), buf0, sem0)
        pltpu.async_copy(_src(1), buf1, sem1)

        def _ring(g, carry):
            for b in range(2):
                q = g * 2 + b

                @pl.when(q < _NCH)
                def _():
                    pltpu.make_async_copy(_src(q), bufs[b], sems[b]).wait()
                    _chunk(q, bufs[b])

                    @pl.when(q + 2 < _NCH)
                    def _():
                        pltpu.async_copy(_src(q + 2), bufs[b], sems[b])
            return carry

        lax.fori_loop(0, (_NCH + 1) // 2, _ring, 0)

        pltpu.sync_copy(m_st, m_hbm.at[pl.ds(row0 * 16, _RPW * 16)])
        pltpu.sync_copy(s_st, s_hbm.at[pl.ds(row0 * 16, _RPW * 16)])
        pltpu.sync_copy(tot_st, tot_hbm.at[pl.ds(row0 * 16, _RPW * 16)])
        pltpu.sync_copy(tv_st, tv_hbm.at[pl.ds(row0 * 16, _RPW * 16)])

    return _sc_stats


def _combine_kernel(am_ref, as_ref, atot_ref, atv_ref,
                    scm_ref, scs_ref, sctot_ref, sctv_ref,
                    tgt_ref, out_ref):
    scm = scm_ref[...]
    mb = jnp.max(scm, axis=1, keepdims=True)
    sb = jnp.sum(scs_ref[...] * jnp.exp(scm - mb), axis=1, keepdims=True)
    ma = am_ref[...]
    mm = jnp.maximum(ma, mb)
    ss = as_ref[...] * jnp.exp(ma - mm) + sb * jnp.exp(mb - mm)
    tot = atot_ref[...] + jnp.sum(sctot_ref[...], axis=1, keepdims=True)
    tv = atv_ref[...] + jnp.sum(sctv_ref[...], axis=1, keepdims=True)
    logz = mm + jnp.log(ss)
    s_row = tot - _V * logz
    logp_t = tv - logz
    loss = -(_FILL * s_row + (1.0 - _EPS - _FILL) * logp_t)
    mask = tgt_ref[...] != 0
    lsum = jnp.sum(jnp.where(mask, loss, 0.0), keepdims=True).reshape(1, 1)
    cnt = jnp.sum(mask.astype(jnp.float32), keepdims=True).reshape(1, 1)
    out_ref[...] = jnp.where(cnt > 0, lsum / jnp.maximum(cnt, 1.0), 0.0)


def kernel(pred, target):
    tgt2 = target.reshape(_N, 1)

    tail = lax.slice(pred, (0, _VSC), (_N, _V))
    scm, scs, sctot, sctv = (a.reshape(_N, 16)
                             for a in _make_sc_stats()(tail, target))

    am, a_s, atot, atv = pl.pallas_call(
        _tc_stats_kernel,
        grid=(_N // _VR,),
        in_specs=[
            pl.BlockSpec((_VR, _VSC), lambda i: (i, 0)),
            pl.BlockSpec((_VR, 1), lambda i: (i, 0)),
        ],
        out_specs=[pl.BlockSpec((_VR, 1), lambda i: (i, 0))] * 4,
        out_shape=[jax.ShapeDtypeStruct((_N, 1), jnp.float32)] * 4,
    )(pred, tgt2)

    out = pl.pallas_call(
        _combine_kernel,
        out_shape=jax.ShapeDtypeStruct((1, 1), jnp.float32),
    )(am, a_s, atot, atv, scm, scs, sctot, sctv, tgt2)
    return out[0, 0]
